# baseline (speedup 1.0000x reference)
import functools

import jax
import jax.numpy as jnp
from jax import lax
from jax.experimental import pallas as pl
from jax.experimental.pallas import tpu as pltpu
from jax.experimental.pallas import tpu_sc as plsc

N = 10000
E = 320000
B = 64
HID = 128
L = 7
NCLS = 10

NW = 32
VW = 64
NPW = 160
NPAD = VW * NPW
EK = 128
EP = E + EK


def _sc_edge_body(h_hbm, e_hbm, src_hbm, dst_hbm, eb_hbm,
                       den_hbm, s_hbm,
                       amax, aden, asum, srcv, dstv, hrows, ebuf, ebv,
                       gsem):
    c = lax.axis_index("c")
    s = lax.axis_index("s")
    wid = s * 2 + c

    pltpu.sync_copy(eb_hbm, ebv)

    for half in range(2):
        vw = wid * 2 + half
        base = vw * NPW

        def zrow(r, carry):
            z = jnp.zeros((16,), jnp.float32)
            for j in range(8):
                sl = pl.ds(j * 16, 16)
                amax[r, sl] = z
                aden[r, sl] = z
                asum[r, sl] = z
            return carry

        lax.fori_loop(0, NPW + 1, zrow, 0)

        ebvec = ebv[pl.ds(vw, 16)]
        lo = ebvec[0]
        hi = ebvec[1]
        a0 = (lo // 8) * 8
        nblk = (hi - a0 + EK - 1) // EK

        def make_blk(pass2):
            def blk(b, carry):
                start = a0 + b * EK
                pltpu.sync_copy(src_hbm.at[pl.ds(start, EK)], srcv)
                pltpu.sync_copy(dst_hbm.at[pl.ds(start, EK)], dstv.at[pl.ds(0, EK)])
                cp = pltpu.async_copy(h_hbm.at[srcv], hrows, gsem)
                pltpu.sync_copy(e_hbm.at[pl.ds(start, EK), :], ebuf)
                cp.wait()

                def edge(i, c2):
                    ge = start + i
                    ok = jnp.logical_and(ge >= lo, ge < hi)
                    dvec = dstv[pl.ds(i, 16)]
                    dl = jnp.where(ok, dvec[0] - base, NPW)
                    for j in range(8):
                        sl = pl.ds(j * 16, 16)
                        m = jnp.maximum(hrows[i, sl] + ebuf[i, sl], 0.0) + 1e-7
                        if not pass2:
                            amax[dl, sl] = jnp.maximum(amax[dl, sl], m)
                        else:
                            num = jnp.exp(m - amax[dl, sl])
                            aden[dl, sl] = aden[dl, sl] + num
                            asum[dl, sl] = asum[dl, sl] + num * m
                    return c2

                lax.fori_loop(0, EK, edge, 0)
                return carry
            return blk

        lax.fori_loop(0, nblk, make_blk(False), 0)
        lax.fori_loop(0, nblk, make_blk(True), 0)

        pltpu.sync_copy(aden.at[pl.ds(0, NPW), :], den_hbm.at[pl.ds(base, NPW), :])
        pltpu.sync_copy(asum.at[pl.ds(0, NPW), :], s_hbm.at[pl.ds(base, NPW), :])


@functools.lru_cache(maxsize=1)
def _get_sc_edge():
    return pl.kernel(
        _sc_edge_body,
        out_type=[
        jax.ShapeDtypeStruct((NPAD, HID), jnp.float32),
        jax.ShapeDtypeStruct((NPAD, HID), jnp.float32),
    ],
    mesh=plsc.VectorSubcoreMesh(core_axis_name="c", subcore_axis_name="s"),
    scratch_types=[
        pltpu.VMEM((NPW + 1, HID), jnp.float32),
        pltpu.VMEM((NPW + 1, HID), jnp.float32),
        pltpu.VMEM((NPW + 1, HID), jnp.float32),
        pltpu.VMEM((EK,), jnp.int32),
        pltpu.VMEM((EK + 16,), jnp.int32),
        pltpu.VMEM((EK, HID), jnp.float32),
        pltpu.VMEM((EK, HID), jnp.float32),
        pltpu.VMEM((VW + 16,), jnp.int32),
        pltpu.SemaphoreType.DMA,
    ],
    )


def _enc_body(idx_ref, tbl_ref, o_ref, *, ncat):
    idx = idx_ref[...]
    blk = idx.shape[0]
    acc = jnp.zeros((blk, ncat), jnp.float32)
    for j in range(idx.shape[1]):
        iot = lax.broadcasted_iota(jnp.int32, (blk, ncat), 1)
        acc = acc + jnp.where(idx[:, j][:, None] == iot, 1.0, 0.0)
    o_ref[...] = jnp.dot(acc, tbl_ref[...], preferred_element_type=jnp.float32, precision=lax.Precision.HIGHEST)


def _encode(idx, table, ncat, blk):
    rows = idx.shape[0]
    grid = (rows + blk - 1) // blk
    return pl.pallas_call(
        functools.partial(_enc_body, ncat=ncat),
        grid=(grid,),
        in_specs=[
            pl.BlockSpec((blk, idx.shape[1]), lambda i: (i, 0)),
            pl.BlockSpec((ncat, HID), lambda i: (0, 0)),
        ],
        out_specs=pl.BlockSpec((blk, HID), lambda i: (i, 0)),
        out_shape=jax.ShapeDtypeStruct((rows, HID), jnp.float32),
    )(idx, table)


def _bn_relu_body(h_ref, g_ref, b_ref, o_ref):
    h = h_ref[...]
    mu = jnp.mean(h, axis=0, keepdims=True)
    var = jnp.mean((h - mu) ** 2, axis=0, keepdims=True)
    hb = g_ref[...] * (h - mu) / jnp.sqrt(var + 1e-5) + b_ref[...]
    o_ref[...] = jnp.maximum(hb, 0.0)


def _bn_relu(h, g, b):
    return pl.pallas_call(
        _bn_relu_body,
        out_shape=jax.ShapeDtypeStruct((N, HID), jnp.float32),
    )(h, g.reshape(1, HID), b.reshape(1, HID))


def _update_body(h2_ref, den_ref, s_ref, w_ref, b_ref, hp_ref, o_ref, *, res):
    agg = s_ref[...] / (den_ref[...] + 1e-16)
    t = h2_ref[...] + agg
    out = jnp.dot(t, w_ref[...], preferred_element_type=jnp.float32, precision=lax.Precision.HIGHEST) + b_ref[...]
    if res:
        out = out + hp_ref[...]
    o_ref[...] = out


def _update(h2, den, s, w, b, hprev, res):
    blk = 1000
    grid = N // blk
    return pl.pallas_call(
        functools.partial(_update_body, res=res),
        grid=(grid,),
        in_specs=[
            pl.BlockSpec((blk, HID), lambda i: (i, 0)),
            pl.BlockSpec((blk, HID), lambda i: (i, 0)),
            pl.BlockSpec((blk, HID), lambda i: (i, 0)),
            pl.BlockSpec((HID, HID), lambda i: (0, 0)),
            pl.BlockSpec((1, HID), lambda i: (0, 0)),
            pl.BlockSpec((blk, HID), lambda i: (i, 0)),
        ],
        out_specs=pl.BlockSpec((blk, HID), lambda i: (i, 0)),
        out_shape=jax.ShapeDtypeStruct((N, HID), jnp.float32),
    )(h2, den, s, w, b.reshape(1, HID), hprev)


def _final_body(h_ref, g_ref, b_ref, bat_ref, wo_ref, bo_ref, o_ref):
    h = h_ref[...]
    mu = jnp.mean(h, axis=0, keepdims=True)
    var = jnp.mean((h - mu) ** 2, axis=0, keepdims=True)
    hb = g_ref[...] * (h - mu) / jnp.sqrt(var + 1e-5) + b_ref[...]
    bat = bat_ref[...]
    iot = lax.broadcasted_iota(jnp.int32, (B, N), 0)
    P = jnp.where(bat == iot, 1.0, 0.0)
    hg = jnp.dot(P, hb, preferred_element_type=jnp.float32, precision=lax.Precision.HIGHEST)
    o_ref[...] = jnp.dot(hg, wo_ref[...], preferred_element_type=jnp.float32, precision=lax.Precision.HIGHEST) + bo_ref[...]


def _final(h, g, b, batch, w_out, b_out):
    return pl.pallas_call(
        _final_body,
        out_shape=jax.ShapeDtypeStruct((B, NCLS), jnp.float32),
    )(h, g.reshape(1, HID), b.reshape(1, HID), batch.reshape(1, N),
      w_out, b_out.reshape(1, NCLS))


def kernel(x, edge_index, edge_attr, batch, atom_table, bond_table, Wg, bg,
           gamma, beta, W_out, b_out):
    src = edge_index[0].astype(jnp.int32)
    dst = edge_index[1].astype(jnp.int32)

    perm = jnp.argsort(dst)
    src_s = jnp.concatenate([src[perm], jnp.zeros((EK,), jnp.int32)])
    dst_s = jnp.concatenate([dst[perm], jnp.zeros((EK,), jnp.int32)])
    eb = jnp.searchsorted(dst_s[:E], jnp.arange(VW + 1, dtype=jnp.int32) * NPW
                          ).astype(jnp.int32)
    eb = jnp.concatenate([eb, jnp.zeros((15,), jnp.int32)])

    xoff = x.astype(jnp.int32) + (jnp.arange(9, dtype=jnp.int32) * 32)[None, :]
    xoff = jnp.pad(xoff, ((0, 0), (0, 7)), constant_values=100000)
    h = _encode(xoff, atom_table, 288, 1000)

    aoff = edge_attr.astype(jnp.int32)[perm] + (jnp.arange(3, dtype=jnp.int32) * 8)[None, :]
    aoff = jnp.pad(aoff, ((0, EK), (0, 5)), constant_values=100000)
    e = _encode(aoff, bond_table, 24, 2048)

    def layer(h_in, h2, l, res):
        den_p, s_p = _get_sc_edge()(h2, e, src_s, dst_s, eb)
        return _update(h2, den_p, s_p, Wg[l], bg[l], h_in, res)

    h = layer(h, h, 0, res=False)
    for l in range(1, L):
        h2 = _bn_relu(h, gamma[l - 1], beta[l - 1])
        h = layer(h, h2, l, res=True)

    return _final(h, gamma[L - 1], beta[L - 1], batch.astype(jnp.int32),
                  W_out, b_out)


# speedup vs baseline: 1.0225x; 1.0225x over previous
import functools

import jax
import jax.numpy as jnp
from jax import lax
from jax.experimental import pallas as pl
from jax.experimental.pallas import tpu as pltpu
from jax.experimental.pallas import tpu_sc as plsc

N = 10000
E = 320000
B = 64
HID = 128
L = 7
NCLS = 10

NW = 32
VW = 64
NPW = 160
NPAD = VW * NPW
EK = 256
EP = E + EK


def _sc_edge_body(h_hbm, e_hbm, src_hbm, dst_hbm, eb_hbm,
                       den_hbm, s_hbm,
                       amax, aden, asum, srcv, dstv, hrows, ebuf, ebv,
                       gsem):
    c = lax.axis_index("c")
    s = lax.axis_index("s")
    wid = s * 2 + c

    pltpu.sync_copy(eb_hbm, ebv)

    for half in range(2):
        vw = wid * 2 + half
        base = vw * NPW

        def zrow(r, carry):
            z = jnp.zeros((16,), jnp.float32)
            for j in range(8):
                sl = pl.ds(j * 16, 16)
                amax[r, sl] = z
                aden[r, sl] = z
                asum[r, sl] = z
            return carry

        lax.fori_loop(0, NPW + 1, zrow, 0)

        ebvec = ebv[pl.ds(vw, 16)]
        lo = ebvec[0]
        hi = ebvec[1]
        a0 = (lo // 8) * 8
        nblk = (hi - a0 + EK - 1) // EK

        def make_blk(pass2):
            def blk(b, carry):
                start = a0 + b * EK
                pltpu.sync_copy(src_hbm.at[pl.ds(start, EK)], srcv)
                pltpu.sync_copy(dst_hbm.at[pl.ds(start, EK)], dstv.at[pl.ds(0, EK)])
                cp = pltpu.async_copy(h_hbm.at[srcv], hrows, gsem)
                pltpu.sync_copy(e_hbm.at[pl.ds(start, EK), :], ebuf)
                cp.wait()

                def edge(i, c2):
                    ge = start + i
                    ok = jnp.logical_and(ge >= lo, ge < hi)
                    dvec = dstv[pl.ds(i, 16)]
                    dl = jnp.where(ok, dvec[0] - base, NPW)
                    for j in range(8):
                        sl = pl.ds(j * 16, 16)
                        m = jnp.maximum(hrows[i, sl] + ebuf[i, sl], 0.0) + 1e-7
                        if not pass2:
                            amax[dl, sl] = jnp.maximum(amax[dl, sl], m)
                        else:
                            num = jnp.exp(m - amax[dl, sl])
                            aden[dl, sl] = aden[dl, sl] + num
                            asum[dl, sl] = asum[dl, sl] + num * m
                    return c2

                lax.fori_loop(0, EK, edge, 0)
                return carry
            return blk

        lax.fori_loop(0, nblk, make_blk(False), 0)
        lax.fori_loop(0, nblk, make_blk(True), 0)

        pltpu.sync_copy(aden.at[pl.ds(0, NPW), :], den_hbm.at[pl.ds(base, NPW), :])
        pltpu.sync_copy(asum.at[pl.ds(0, NPW), :], s_hbm.at[pl.ds(base, NPW), :])


@functools.lru_cache(maxsize=1)
def _get_sc_edge():
    return pl.kernel(
        _sc_edge_body,
        out_type=[
        jax.ShapeDtypeStruct((NPAD, HID), jnp.float32),
        jax.ShapeDtypeStruct((NPAD, HID), jnp.float32),
    ],
    mesh=plsc.VectorSubcoreMesh(core_axis_name="c", subcore_axis_name="s"),
    scratch_types=[
        pltpu.VMEM((NPW + 1, HID), jnp.float32),
        pltpu.VMEM((NPW + 1, HID), jnp.float32),
        pltpu.VMEM((NPW + 1, HID), jnp.float32),
        pltpu.VMEM((EK,), jnp.int32),
        pltpu.VMEM((EK + 16,), jnp.int32),
        pltpu.VMEM((EK, HID), jnp.float32),
        pltpu.VMEM((EK, HID), jnp.float32),
        pltpu.VMEM((VW + 16,), jnp.int32),
        pltpu.SemaphoreType.DMA,
    ],
    )


def _enc_body(idx_ref, tbl_ref, o_ref, *, ncat):
    idx = idx_ref[...]
    blk = idx.shape[0]
    acc = jnp.zeros((blk, ncat), jnp.float32)
    for j in range(idx.shape[1]):
        iot = lax.broadcasted_iota(jnp.int32, (blk, ncat), 1)
        acc = acc + jnp.where(idx[:, j][:, None] == iot, 1.0, 0.0)
    o_ref[...] = jnp.dot(acc, tbl_ref[...], preferred_element_type=jnp.float32, precision=lax.Precision.HIGHEST)


def _encode(idx, table, ncat, blk):
    rows = idx.shape[0]
    grid = (rows + blk - 1) // blk
    return pl.pallas_call(
        functools.partial(_enc_body, ncat=ncat),
        grid=(grid,),
        in_specs=[
            pl.BlockSpec((blk, idx.shape[1]), lambda i: (i, 0)),
            pl.BlockSpec((ncat, HID), lambda i: (0, 0)),
        ],
        out_specs=pl.BlockSpec((blk, HID), lambda i: (i, 0)),
        out_shape=jax.ShapeDtypeStruct((rows, HID), jnp.float32),
    )(idx, table)


def _bn_relu_body(h_ref, g_ref, b_ref, o_ref):
    h = h_ref[...]
    mu = jnp.mean(h, axis=0, keepdims=True)
    var = jnp.mean((h - mu) ** 2, axis=0, keepdims=True)
    hb = g_ref[...] * (h - mu) / jnp.sqrt(var + 1e-5) + b_ref[...]
    o_ref[...] = jnp.maximum(hb, 0.0)


def _bn_relu(h, g, b):
    return pl.pallas_call(
        _bn_relu_body,
        out_shape=jax.ShapeDtypeStruct((N, HID), jnp.float32),
    )(h, g.reshape(1, HID), b.reshape(1, HID))


def _update_body(h2_ref, den_ref, s_ref, w_ref, b_ref, hp_ref, o_ref, *, res):
    agg = s_ref[...] / (den_ref[...] + 1e-16)
    t = h2_ref[...] + agg
    out = jnp.dot(t, w_ref[...], preferred_element_type=jnp.float32, precision=lax.Precision.HIGHEST) + b_ref[...]
    if res:
        out = out + hp_ref[...]
    o_ref[...] = out


def _update(h2, den, s, w, b, hprev, res):
    blk = 1000
    grid = N // blk
    return pl.pallas_call(
        functools.partial(_update_body, res=res),
        grid=(grid,),
        in_specs=[
            pl.BlockSpec((blk, HID), lambda i: (i, 0)),
            pl.BlockSpec((blk, HID), lambda i: (i, 0)),
            pl.BlockSpec((blk, HID), lambda i: (i, 0)),
            pl.BlockSpec((HID, HID), lambda i: (0, 0)),
            pl.BlockSpec((1, HID), lambda i: (0, 0)),
            pl.BlockSpec((blk, HID), lambda i: (i, 0)),
        ],
        out_specs=pl.BlockSpec((blk, HID), lambda i: (i, 0)),
        out_shape=jax.ShapeDtypeStruct((N, HID), jnp.float32),
    )(h2, den, s, w, b.reshape(1, HID), hprev)


def _final_body(h_ref, g_ref, b_ref, bat_ref, wo_ref, bo_ref, o_ref):
    h = h_ref[...]
    mu = jnp.mean(h, axis=0, keepdims=True)
    var = jnp.mean((h - mu) ** 2, axis=0, keepdims=True)
    hb = g_ref[...] * (h - mu) / jnp.sqrt(var + 1e-5) + b_ref[...]
    bat = bat_ref[...]
    iot = lax.broadcasted_iota(jnp.int32, (B, N), 0)
    P = jnp.where(bat == iot, 1.0, 0.0)
    hg = jnp.dot(P, hb, preferred_element_type=jnp.float32, precision=lax.Precision.HIGHEST)
    o_ref[...] = jnp.dot(hg, wo_ref[...], preferred_element_type=jnp.float32, precision=lax.Precision.HIGHEST) + bo_ref[...]


def _final(h, g, b, batch, w_out, b_out):
    return pl.pallas_call(
        _final_body,
        out_shape=jax.ShapeDtypeStruct((B, NCLS), jnp.float32),
    )(h, g.reshape(1, HID), b.reshape(1, HID), batch.reshape(1, N),
      w_out, b_out.reshape(1, NCLS))


def kernel(x, edge_index, edge_attr, batch, atom_table, bond_table, Wg, bg,
           gamma, beta, W_out, b_out):
    src = edge_index[0].astype(jnp.int32)
    dst = edge_index[1].astype(jnp.int32)

    perm = jnp.argsort(dst)
    src_s = jnp.concatenate([src[perm], jnp.zeros((EK,), jnp.int32)])
    dst_s = jnp.concatenate([dst[perm], jnp.zeros((EK,), jnp.int32)])
    eb = jnp.searchsorted(dst_s[:E], jnp.arange(VW + 1, dtype=jnp.int32) * NPW
                          ).astype(jnp.int32)
    eb = jnp.concatenate([eb, jnp.zeros((15,), jnp.int32)])

    xoff = x.astype(jnp.int32) + (jnp.arange(9, dtype=jnp.int32) * 32)[None, :]
    xoff = jnp.pad(xoff, ((0, 0), (0, 7)), constant_values=100000)
    h = _encode(xoff, atom_table, 288, 1000)

    aoff = edge_attr.astype(jnp.int32)[perm] + (jnp.arange(3, dtype=jnp.int32) * 8)[None, :]
    aoff = jnp.pad(aoff, ((0, EK), (0, 5)), constant_values=100000)
    e = _encode(aoff, bond_table, 24, 2048)

    def layer(h_in, h2, l, res):
        den_p, s_p = _get_sc_edge()(h2, e, src_s, dst_s, eb)
        return _update(h2, den_p, s_p, Wg[l], bg[l], h_in, res)

    h = layer(h, h, 0, res=False)
    for l in range(1, L):
        h2 = _bn_relu(h, gamma[l - 1], beta[l - 1])
        h = layer(h, h2, l, res=True)

    return _final(h, gamma[L - 1], beta[L - 1], batch.astype(jnp.int32),
                  W_out, b_out)
